# Initial kernel scaffold; baseline (speedup 1.0000x reference)
import functools

import jax
import jax.numpy as jnp
from jax import lax
from jax.experimental import pallas as pl
from jax.experimental.pallas import tpu as pltpu
from jax.experimental.pallas import tpu_sc as plsc

NC = 2
NS = 16
NW = NC * NS
L = 16
CHUNK = 128

_f32 = jnp.float32


def _sds(shape):
    return jax.ShapeDtypeStruct(shape, _f32)


def _sc_scatter_rows(g, row3, col3, npad):
    nch = row3.shape[1]
    rpt = npad // NS
    mesh = plsc.VectorSubcoreMesh(core_axis_name="c", subcore_axis_name="s")

    @functools.partial(
        pl.kernel,
        out_type=_sds((NC, npad, L)),
        mesh=mesh,
        scratch_types=[
            pltpu.VMEM((nch, CHUNK), jnp.int32),
            pltpu.VMEM((nch, CHUNK), jnp.int32),
            pltpu.VMEM((CHUNK, L), _f32),
            pltpu.VMEM((rpt, L), _f32),
            pltpu.VMEM_SHARED((npad, L), _f32),
            pltpu.SemaphoreType.DMA,
        ],
    )
    def hop(g_hbm, row_hbm, col_hbm, out_hbm, row_v, col_v, msg_v, z_v,
            acc_sh, sem):
        cid = lax.axis_index("c")
        sid = lax.axis_index("s")
        wid = sid * NC + cid

        pltpu.sync_copy(row_hbm.at[wid], row_v)
        pltpu.sync_copy(col_hbm.at[wid], col_v)

        zero = jnp.zeros((L,), _f32)

        @pl.loop(0, rpt)
        def _(i):
            z_v[i] = zero

        pltpu.sync_copy(z_v, acc_sh.at[pl.ds(sid * rpt, rpt)])
        plsc.subcore_barrier()

        @pl.loop(0, nch)
        def _(j):
            pltpu.async_copy(g_hbm.at[row_v.at[j]], msg_v, sem).wait()
            pltpu.sync_copy(msg_v, acc_sh.at[col_v.at[j]], add=True)

        plsc.subcore_barrier()
        pltpu.sync_copy(acc_sh.at[pl.ds(sid * rpt, rpt)],
                        out_hbm.at[cid, pl.ds(sid * rpt, rpt)])

    return hop(g, row3, col3)


def _sc_degree(col3, npad):
    nch = col3.shape[1]
    rpt = npad // NS
    mesh = plsc.VectorSubcoreMesh(core_axis_name="c", subcore_axis_name="s")

    @functools.partial(
        pl.kernel,
        out_type=_sds((NC, npad, L)),
        mesh=mesh,
        scratch_types=[
            pltpu.VMEM((nch, CHUNK), jnp.int32),
            pltpu.VMEM((CHUNK, L), _f32),
            pltpu.VMEM((rpt, L), _f32),
            pltpu.VMEM_SHARED((npad, L), _f32),
        ],
    )
    def deg(col_hbm, out_hbm, col_v, one_v, z_v, acc_sh):
        cid = lax.axis_index("c")
        sid = lax.axis_index("s")
        wid = sid * NC + cid

        pltpu.sync_copy(col_hbm.at[wid], col_v)

        zero = jnp.zeros((L,), _f32)
        one = jnp.ones((L,), _f32)

        @pl.loop(0, CHUNK)
        def _(i):
            one_v[i] = one

        @pl.loop(0, rpt)
        def _(i):
            z_v[i] = zero

        pltpu.sync_copy(z_v, acc_sh.at[pl.ds(sid * rpt, rpt)])
        plsc.subcore_barrier()

        @pl.loop(0, nch)
        def _(j):
            pltpu.sync_copy(one_v, acc_sh.at[col_v.at[j]], add=True)

        plsc.subcore_barrier()
        pltpu.sync_copy(acc_sh.at[pl.ds(sid * rpt, rpt)],
                        out_hbm.at[cid, pl.ds(sid * rpt, rpt)])

    return deg(col3)


def _tc_project(x, w):
    def mm(x_ref, w_ref, o_ref):
        o_ref[...] = lax.dot_general(
            x_ref[...], w_ref[...], (((1,), (1,)), ((), ())),
            preferred_element_type=_f32)

    return pl.pallas_call(
        mm, out_shape=_sds((x.shape[0], w.shape[0])))(x, w)


def _tc_norm_init(xwp, degp):
    def f(xw_ref, dp_ref, g0_ref, dv_ref):
        dv = lax.rsqrt(1.0 + dp_ref[0] + dp_ref[1])
        dv_ref[...] = dv
        g0_ref[...] = dv * xw_ref[...]

    return pl.pallas_call(
        f, out_shape=(_sds(xwp.shape), _sds(xwp.shape)))(xwp, degp)


def _tc_mid(s, g, dv):
    def f(s_ref, g_ref, dv_ref, o_ref):
        d = dv_ref[...]
        o_ref[...] = d * d * (s_ref[0] + s_ref[1] + g_ref[...])

    return pl.pallas_call(f, out_shape=_sds(g.shape))(s, g, dv)


def _tc_final(s, g, dv, b2):
    def f(s_ref, g_ref, dv_ref, b_ref, o_ref):
        h = dv_ref[...] * (s_ref[0] + s_ref[1] + g_ref[...])
        logits = h + b_ref[...]
        m = jnp.max(logits, axis=1, keepdims=True)
        e = jnp.exp(logits - m)
        z = jnp.sum(e, axis=1, keepdims=True)
        o_ref[...] = logits - m - jnp.log(z)

    return pl.pallas_call(f, out_shape=_sds(g.shape))(s, g, dv, b2)


def kernel(x, edge_index, W, b):
    n, d = x.shape
    c = W.shape[0]
    e = edge_index.shape[1]
    assert c == L

    npad = ((n + CHUNK - 1) // CHUNK) * CHUNK
    nch = (e + NW * CHUNK - 1) // (NW * CHUNK)
    epad = NW * nch * CHUNK

    row3 = jnp.pad(edge_index[0], (0, epad - e)).reshape(NW, nch, CHUNK)
    col3 = jnp.pad(edge_index[1], (0, epad - e),
                   constant_values=n).reshape(NW, nch, CHUNK)

    degp = _sc_degree(col3, npad)
    xw = _tc_project(x, W)
    xwp = jnp.pad(xw, ((0, npad - n), (0, 0)))
    g0, dv = _tc_norm_init(xwp, degp)
    s1 = _sc_scatter_rows(g0, row3, col3, npad)
    g1 = _tc_mid(s1, g0, dv)
    s2 = _sc_scatter_rows(g1, row3, col3, npad)
    out = _tc_final(s2, g1, dv, b.reshape(1, L))
    return out[:n]


# speedup vs baseline: 33.0750x; 33.0750x over previous
import functools

import jax
import jax.numpy as jnp
from jax import lax
from jax.experimental import pallas as pl
from jax.experimental.pallas import tpu as pltpu
from jax.experimental.pallas import tpu_sc as plsc

NC = 2
NS = 16
NW = NC * NS
L = 16
CHUNK = 128

_f32 = jnp.float32


def _sds(shape):
    return jax.ShapeDtypeStruct(shape, _f32)


def _sc_scatter_rows(g, row3, col3, npad):
    nch = row3.shape[1]
    rpt = npad // NS
    mesh = plsc.VectorSubcoreMesh(core_axis_name="c", subcore_axis_name="s")

    @functools.partial(
        pl.kernel,
        out_type=_sds((NC, npad, L)),
        mesh=mesh,
        scratch_types=[
            pltpu.VMEM((nch, CHUNK), jnp.int32),
            pltpu.VMEM((nch, CHUNK), jnp.int32),
            pltpu.VMEM((CHUNK, L), _f32),
            pltpu.VMEM((rpt, L), _f32),
            pltpu.VMEM_SHARED((npad, L), _f32),
            pltpu.SemaphoreType.DMA,
        ],
        compiler_params=pltpu.CompilerParams(use_tc_tiling_on_sc=False),
    )
    def hop(g_hbm, row_hbm, col_hbm, out_hbm, row_v, col_v, msg_v, z_v,
            acc_sh, sem):
        cid = lax.axis_index("c")
        sid = lax.axis_index("s")
        wid = sid * NC + cid

        pltpu.sync_copy(row_hbm.at[wid], row_v)
        pltpu.sync_copy(col_hbm.at[wid], col_v)

        zero = jnp.zeros((L,), _f32)

        @pl.loop(0, rpt)
        def _(i):
            z_v[i] = zero

        pltpu.sync_copy(z_v, acc_sh.at[pl.ds(sid * rpt, rpt)])
        plsc.subcore_barrier()

        @pl.loop(0, nch)
        def _(j):
            pltpu.async_copy(g_hbm.at[row_v.at[j]], msg_v, sem).wait()
            pltpu.sync_copy(msg_v, acc_sh.at[col_v.at[j]], add=True)

        plsc.subcore_barrier()
        pltpu.sync_copy(acc_sh.at[pl.ds(sid * rpt, rpt)],
                        out_hbm.at[cid, pl.ds(sid * rpt, rpt)])

    return hop(g, row3, col3)


def _sc_degree(col3, npad):
    nch = col3.shape[1]
    rpt = npad // NS
    mesh = plsc.VectorSubcoreMesh(core_axis_name="c", subcore_axis_name="s")

    @functools.partial(
        pl.kernel,
        out_type=_sds((NC, npad, L)),
        mesh=mesh,
        scratch_types=[
            pltpu.VMEM((nch, CHUNK), jnp.int32),
            pltpu.VMEM((CHUNK, L), _f32),
            pltpu.VMEM((rpt, L), _f32),
            pltpu.VMEM_SHARED((npad, L), _f32),
        ],
        compiler_params=pltpu.CompilerParams(use_tc_tiling_on_sc=False),
    )
    def deg(col_hbm, out_hbm, col_v, one_v, z_v, acc_sh):
        cid = lax.axis_index("c")
        sid = lax.axis_index("s")
        wid = sid * NC + cid

        pltpu.sync_copy(col_hbm.at[wid], col_v)

        zero = jnp.zeros((L,), _f32)
        one = jnp.ones((L,), _f32)

        @pl.loop(0, CHUNK)
        def _(i):
            one_v[i] = one

        @pl.loop(0, rpt)
        def _(i):
            z_v[i] = zero

        pltpu.sync_copy(z_v, acc_sh.at[pl.ds(sid * rpt, rpt)])
        plsc.subcore_barrier()

        @pl.loop(0, nch)
        def _(j):
            pltpu.sync_copy(one_v, acc_sh.at[col_v.at[j]], add=True)

        plsc.subcore_barrier()
        pltpu.sync_copy(acc_sh.at[pl.ds(sid * rpt, rpt)],
                        out_hbm.at[cid, pl.ds(sid * rpt, rpt)])

    return deg(col3)


def _tc_project(x, w):
    def mm(x_ref, w_ref, o_ref):
        o_ref[...] = lax.dot_general(
            x_ref[...], w_ref[...], (((1,), (1,)), ((), ())),
            preferred_element_type=_f32)

    return pl.pallas_call(
        mm, out_shape=_sds((x.shape[0], w.shape[0])))(x, w)


def _tc_norm_init(xwp, degp):
    def f(xw_ref, dp_ref, g0_ref, dv_ref):
        dv = lax.rsqrt(1.0 + dp_ref[0] + dp_ref[1])
        dv_ref[...] = dv
        g0_ref[...] = dv * xw_ref[...]

    return pl.pallas_call(
        f, out_shape=(_sds(xwp.shape), _sds(xwp.shape)))(xwp, degp)


def _tc_mid(s, g, dv):
    def f(s_ref, g_ref, dv_ref, o_ref):
        d = dv_ref[...]
        o_ref[...] = d * d * (s_ref[0] + s_ref[1] + g_ref[...])

    return pl.pallas_call(f, out_shape=_sds(g.shape))(s, g, dv)


def _tc_final(s, g, dv, b2):
    def f(s_ref, g_ref, dv_ref, b_ref, o_ref):
        h = dv_ref[...] * (s_ref[0] + s_ref[1] + g_ref[...])
        logits = h + b_ref[...]
        m = jnp.max(logits, axis=1, keepdims=True)
        e = jnp.exp(logits - m)
        z = jnp.sum(e, axis=1, keepdims=True)
        o_ref[...] = logits - m - jnp.log(z)

    return pl.pallas_call(f, out_shape=_sds(g.shape))(s, g, dv, b2)


def kernel(x, edge_index, W, b):
    n, d = x.shape
    c = W.shape[0]
    e = edge_index.shape[1]
    assert c == L

    npad = ((n + CHUNK - 1) // CHUNK) * CHUNK
    nch = (e + NW * CHUNK - 1) // (NW * CHUNK)
    epad = NW * nch * CHUNK

    row3 = jnp.pad(edge_index[0], (0, epad - e)).reshape(NW, nch, CHUNK)
    col3 = jnp.pad(edge_index[1], (0, epad - e),
                   constant_values=n).reshape(NW, nch, CHUNK)

    degp = _sc_degree(col3, npad)
    xw = _tc_project(x, W)
    xwp = jnp.pad(xw, ((0, npad - n), (0, 0)))
    g0, dv = _tc_norm_init(xwp, degp)
    s1 = _sc_scatter_rows(g0, row3, col3, npad)
    g1 = _tc_mid(s1, g0, dv)
    s2 = _sc_scatter_rows(g1, row3, col3, npad)
    out = _tc_final(s2, g1, dv, b.reshape(1, L))
    return out[:n]


# speedup vs baseline: 37.7377x; 1.1410x over previous
import functools

import jax
import jax.numpy as jnp
from jax import lax
from jax.experimental import pallas as pl
from jax.experimental.pallas import tpu as pltpu
from jax.experimental.pallas import tpu_sc as plsc

NC = 2
NS = 16
NW = NC * NS
L = 16
CHUNK = 128

_f32 = jnp.float32


def _sds(shape):
    return jax.ShapeDtypeStruct(shape, _f32)


def _sc_scatter_rows(g, row3, col3, npad):
    nch = row3.shape[1]
    rpt = npad // NS
    mesh = plsc.VectorSubcoreMesh(core_axis_name="c", subcore_axis_name="s")

    @functools.partial(
        pl.kernel,
        out_type=_sds((NC, npad, L)),
        mesh=mesh,
        scratch_types=[
            pltpu.VMEM((nch, CHUNK), jnp.int32),
            pltpu.VMEM((nch, CHUNK), jnp.int32),
            pltpu.VMEM((CHUNK, L), _f32),
            pltpu.VMEM((CHUNK, L), _f32),
            pltpu.VMEM((rpt, L), _f32),
            pltpu.VMEM_SHARED((npad, L), _f32),
            pltpu.SemaphoreType.DMA,
            pltpu.SemaphoreType.DMA,
        ],
        compiler_params=pltpu.CompilerParams(use_tc_tiling_on_sc=False),
    )
    def hop(g_hbm, row_hbm, col_hbm, out_hbm, row_v, col_v, msg_a, msg_b,
            z_v, acc_sh, sem_a, sem_b):
        cid = lax.axis_index("c")
        sid = lax.axis_index("s")
        wid = sid * NC + cid

        pltpu.sync_copy(row_hbm.at[wid], row_v)
        pltpu.sync_copy(col_hbm.at[wid], col_v)

        zero = jnp.zeros((L,), _f32)

        @pl.loop(0, rpt)
        def _(i):
            z_v[i] = zero

        pltpu.sync_copy(z_v, acc_sh.at[pl.ds(sid * rpt, rpt)])
        plsc.subcore_barrier()

        last = nch - 1
        pltpu.async_copy(g_hbm.at[row_v.at[0]], msg_a, sem_a)

        @pl.loop(0, nch // 2)
        def _(jo):
            ja = 2 * jo
            jb = ja + 1
            pltpu.async_copy(g_hbm.at[row_v.at[jb]], msg_b, sem_b)
            pltpu.make_async_copy(g_hbm.at[row_v.at[ja]], msg_a, sem_a).wait()
            pltpu.sync_copy(msg_a, acc_sh.at[col_v.at[ja]], add=True)
            ka = jnp.minimum(ja + 2, last)
            pltpu.async_copy(g_hbm.at[row_v.at[ka]], msg_a, sem_a)
            pltpu.make_async_copy(g_hbm.at[row_v.at[jb]], msg_b, sem_b).wait()
            pltpu.sync_copy(msg_b, acc_sh.at[col_v.at[jb]], add=True)

        pltpu.make_async_copy(g_hbm.at[row_v.at[last]], msg_a, sem_a).wait()
        plsc.subcore_barrier()
        pltpu.sync_copy(acc_sh.at[pl.ds(sid * rpt, rpt)],
                        out_hbm.at[cid, pl.ds(sid * rpt, rpt)])

    return hop(g, row3, col3)


def _sc_degree(col3, npad):
    nch = col3.shape[1]
    rpt = npad // NS
    mesh = plsc.VectorSubcoreMesh(core_axis_name="c", subcore_axis_name="s")

    @functools.partial(
        pl.kernel,
        out_type=_sds((NC, npad, L)),
        mesh=mesh,
        scratch_types=[
            pltpu.VMEM((nch, CHUNK), jnp.int32),
            pltpu.VMEM((CHUNK, L), _f32),
            pltpu.VMEM((rpt, L), _f32),
            pltpu.VMEM_SHARED((npad, L), _f32),
        ],
        compiler_params=pltpu.CompilerParams(use_tc_tiling_on_sc=False),
    )
    def deg(col_hbm, out_hbm, col_v, one_v, z_v, acc_sh):
        cid = lax.axis_index("c")
        sid = lax.axis_index("s")
        wid = sid * NC + cid

        pltpu.sync_copy(col_hbm.at[wid], col_v)

        zero = jnp.zeros((L,), _f32)
        one = jnp.ones((L,), _f32)

        @pl.loop(0, CHUNK)
        def _(i):
            one_v[i] = one

        @pl.loop(0, rpt)
        def _(i):
            z_v[i] = zero

        pltpu.sync_copy(z_v, acc_sh.at[pl.ds(sid * rpt, rpt)])
        plsc.subcore_barrier()

        @pl.loop(0, nch)
        def _(j):
            pltpu.sync_copy(one_v, acc_sh.at[col_v.at[j]], add=True)

        plsc.subcore_barrier()
        pltpu.sync_copy(acc_sh.at[pl.ds(sid * rpt, rpt)],
                        out_hbm.at[cid, pl.ds(sid * rpt, rpt)])

    return deg(col3)


def _tc_project_norm(x, w, degp, npad):
    n = x.shape[0]

    def f(x_ref, w_ref, dp_ref, g0_ref, dv_ref):
        dv = lax.rsqrt(1.0 + dp_ref[0] + dp_ref[1])
        dv_ref[...] = dv
        xw = lax.dot_general(
            x_ref[...], w_ref[...], (((1,), (1,)), ((), ())),
            preferred_element_type=_f32)
        g0_ref[:n] = dv[:n] * xw
        g0_ref[n:] = jnp.zeros((npad - n, L), _f32)

    return pl.pallas_call(
        f, out_shape=(_sds((npad, L)), _sds((npad, L))))(x, w, degp)


def _tc_mid(s, g, dv):
    def f(s_ref, g_ref, dv_ref, o_ref):
        d = dv_ref[...]
        o_ref[...] = d * d * (s_ref[0] + s_ref[1] + g_ref[...])

    return pl.pallas_call(f, out_shape=_sds(g.shape))(s, g, dv)


def _tc_final(s, g, dv, b2):
    def f(s_ref, g_ref, dv_ref, b_ref, o_ref):
        h = dv_ref[...] * (s_ref[0] + s_ref[1] + g_ref[...])
        logits = h + b_ref[...]
        m = jnp.max(logits, axis=1, keepdims=True)
        e = jnp.exp(logits - m)
        z = jnp.sum(e, axis=1, keepdims=True)
        o_ref[...] = logits - m - jnp.log(z)

    return pl.pallas_call(f, out_shape=_sds(g.shape))(s, g, dv, b2)


def kernel(x, edge_index, W, b):
    n, d = x.shape
    c = W.shape[0]
    e = edge_index.shape[1]
    assert c == L

    npad = ((n + CHUNK - 1) // CHUNK) * CHUNK
    nch = (e + NW * CHUNK - 1) // (NW * CHUNK)
    nch = ((nch + 1) // 2) * 2
    epad = NW * nch * CHUNK

    row3 = jnp.pad(edge_index[0], (0, epad - e)).reshape(NW, nch, CHUNK)
    col3 = jnp.pad(edge_index[1], (0, epad - e),
                   constant_values=n).reshape(NW, nch, CHUNK)

    degp = _sc_degree(col3, npad)
    g0, dv = _tc_project_norm(x, W, degp, npad)
    s1 = _sc_scatter_rows(g0, row3, col3, npad)
    g1 = _tc_mid(s1, g0, dv)
    s2 = _sc_scatter_rows(g1, row3, col3, npad)
    out = _tc_final(s2, g1, dv, b.reshape(1, L))
    return out[:n]


# speedup vs baseline: 39.0363x; 1.0344x over previous
import functools

import jax
import jax.numpy as jnp
from jax import lax
from jax.experimental import pallas as pl
from jax.experimental.pallas import tpu as pltpu
from jax.experimental.pallas import tpu_sc as plsc

NC = 2
NS = 16
NW = NC * NS
L = 16
CHUNK = 128

_f32 = jnp.float32


def _sds(shape):
    return jax.ShapeDtypeStruct(shape, _f32)


def _sc_scatter_rows(g, row3, col3, npad):
    nch = row3.shape[1]
    rpt = npad // NS
    mesh = plsc.VectorSubcoreMesh(core_axis_name="c", subcore_axis_name="s")

    @functools.partial(
        pl.kernel,
        out_type=_sds((NC, npad, L)),
        mesh=mesh,
        scratch_types=[
            pltpu.VMEM((nch, CHUNK), jnp.int32),
            pltpu.VMEM((nch, CHUNK), jnp.int32),
            [pltpu.VMEM((CHUNK, L), _f32)] * 4,
            pltpu.VMEM((rpt, L), _f32),
            pltpu.VMEM_SHARED((npad, L), _f32),
            [pltpu.SemaphoreType.DMA] * 4,
            [pltpu.SemaphoreType.DMA] * 4,
        ],
        compiler_params=pltpu.CompilerParams(use_tc_tiling_on_sc=False),
    )
    def hop(g_hbm, row_hbm, col_hbm, out_hbm, row_v, col_v, msg, z_v,
            acc_sh, gsem, ssem):
        cid = lax.axis_index("c")
        sid = lax.axis_index("s")
        wid = sid * NC + cid

        pltpu.sync_copy(row_hbm.at[wid], row_v)
        pltpu.sync_copy(col_hbm.at[wid], col_v)

        zero = jnp.zeros((L,), _f32)

        @pl.loop(0, rpt)
        def _(i):
            z_v[i] = zero

        pltpu.sync_copy(z_v, acc_sh.at[pl.ds(sid * rpt, rpt)])
        plsc.subcore_barrier()

        def g_issue(b, j):
            pltpu.async_copy(g_hbm.at[row_v.at[j]], msg[b], gsem[b])

        def g_wait(b, j):
            pltpu.make_async_copy(g_hbm.at[row_v.at[j]], msg[b],
                                  gsem[b]).wait()

        def s_issue(b, j):
            pltpu.async_copy(msg[b], acc_sh.at[col_v.at[j]], ssem[b],
                             add=True)

        def s_wait(b, j):
            pltpu.make_async_copy(msg[b], acc_sh.at[col_v.at[j]],
                                  ssem[b]).wait()

        last = nch - 1
        g_issue(0, 0)
        g_issue(1, 1)
        g_wait(0, 0); s_issue(0, 0); g_issue(2, 2)
        g_wait(1, 1); s_issue(1, 1); g_issue(3, 3)
        g_wait(2, 2); s_issue(2, 2); s_wait(0, 0); g_issue(0, 4)
        g_wait(3, 3); s_issue(3, 3); s_wait(1, 1); g_issue(1, 5)

        @pl.loop(1, nch // 4)
        def _(jo):
            j0 = 4 * jo
            for b in range(4):
                j = j0 + b
                g_wait(b, j)
                s_issue(b, j)
                s_wait((b + 2) % 4, j - 2)
                g_issue((b + 2) % 4, jnp.minimum(j + 2, last))

        s_wait(2, nch - 2)
        s_wait(3, nch - 1)
        g_wait(0, last)
        g_wait(1, last)
        plsc.subcore_barrier()
        pltpu.sync_copy(acc_sh.at[pl.ds(sid * rpt, rpt)],
                        out_hbm.at[cid, pl.ds(sid * rpt, rpt)])

    return hop(g, row3, col3)


def _sc_degree(col3, npad):
    nch = col3.shape[1]
    rpt = npad // NS
    mesh = plsc.VectorSubcoreMesh(core_axis_name="c", subcore_axis_name="s")

    @functools.partial(
        pl.kernel,
        out_type=_sds((NC, npad, L)),
        mesh=mesh,
        scratch_types=[
            pltpu.VMEM((nch, CHUNK), jnp.int32),
            pltpu.VMEM((CHUNK, L), _f32),
            pltpu.VMEM((rpt, L), _f32),
            pltpu.VMEM_SHARED((npad, L), _f32),
            pltpu.SemaphoreType.DMA,
        ],
        compiler_params=pltpu.CompilerParams(use_tc_tiling_on_sc=False),
    )
    def deg(col_hbm, out_hbm, col_v, one_v, z_v, acc_sh, sem):
        cid = lax.axis_index("c")
        sid = lax.axis_index("s")
        wid = sid * NC + cid

        pltpu.sync_copy(col_hbm.at[wid], col_v)

        zero = jnp.zeros((L,), _f32)
        one = jnp.ones((L,), _f32)

        @pl.loop(0, CHUNK)
        def _(i):
            one_v[i] = one

        @pl.loop(0, rpt)
        def _(i):
            z_v[i] = zero

        pltpu.sync_copy(z_v, acc_sh.at[pl.ds(sid * rpt, rpt)])
        plsc.subcore_barrier()

        @pl.loop(0, nch)
        def _(j):
            pltpu.async_copy(one_v, acc_sh.at[col_v.at[j]], sem, add=True)

        @pl.loop(0, nch)
        def _(j):
            pltpu.make_async_copy(one_v, acc_sh.at[col_v.at[j]], sem).wait()

        plsc.subcore_barrier()
        pltpu.sync_copy(acc_sh.at[pl.ds(sid * rpt, rpt)],
                        out_hbm.at[cid, pl.ds(sid * rpt, rpt)])

    return deg(col3)


def _tc_project_norm(x, w, degp, npad):
    n = x.shape[0]

    def f(x_ref, w_ref, dp_ref, g0_ref, dv_ref):
        dv = lax.rsqrt(1.0 + dp_ref[0] + dp_ref[1])
        dv_ref[...] = dv
        xw = lax.dot_general(
            x_ref[...], w_ref[...], (((1,), (1,)), ((), ())),
            preferred_element_type=_f32)
        g0_ref[:n] = dv[:n] * xw
        g0_ref[n:] = jnp.zeros((npad - n, L), _f32)

    return pl.pallas_call(
        f, out_shape=(_sds((npad, L)), _sds((npad, L))))(x, w, degp)


def _tc_mid(s, g, dv):
    def f(s_ref, g_ref, dv_ref, o_ref):
        d = dv_ref[...]
        o_ref[...] = d * d * (s_ref[0] + s_ref[1] + g_ref[...])

    return pl.pallas_call(f, out_shape=_sds(g.shape))(s, g, dv)


def _tc_final(s, g, dv, b2):
    def f(s_ref, g_ref, dv_ref, b_ref, o_ref):
        h = dv_ref[...] * (s_ref[0] + s_ref[1] + g_ref[...])
        logits = h + b_ref[...]
        m = jnp.max(logits, axis=1, keepdims=True)
        e = jnp.exp(logits - m)
        z = jnp.sum(e, axis=1, keepdims=True)
        o_ref[...] = logits - m - jnp.log(z)

    return pl.pallas_call(f, out_shape=_sds(g.shape))(s, g, dv, b2)


def kernel(x, edge_index, W, b):
    n, d = x.shape
    c = W.shape[0]
    e = edge_index.shape[1]
    assert c == L

    npad = ((n + CHUNK - 1) // CHUNK) * CHUNK
    nch = (e + NW * CHUNK - 1) // (NW * CHUNK)
    nch = max(8, ((nch + 3) // 4) * 4)
    epad = NW * nch * CHUNK

    row3 = jnp.pad(edge_index[0], (0, epad - e)).reshape(NW, nch, CHUNK)
    col3 = jnp.pad(edge_index[1], (0, epad - e),
                   constant_values=n).reshape(NW, nch, CHUNK)

    degp = _sc_degree(col3, npad)
    g0, dv = _tc_project_norm(x, W, degp, npad)
    s1 = _sc_scatter_rows(g0, row3, col3, npad)
    g1 = _tc_mid(s1, g0, dv)
    s2 = _sc_scatter_rows(g1, row3, col3, npad)
    out = _tc_final(s2, g1, dv, b.reshape(1, L))
    return out[:n]


# speedup vs baseline: 62.9894x; 1.6136x over previous
import functools

import jax
import jax.numpy as jnp
from jax import lax
from jax.experimental import pallas as pl
from jax.experimental.pallas import tpu as pltpu
from jax.experimental.pallas import tpu_sc as plsc

NC = 2
NS = 16
NW = NC * NS
L = 16
CHUNK = 128

_f32 = jnp.float32


def _sds(shape):
    return jax.ShapeDtypeStruct(shape, _f32)


def _rsqrt_sc(d):
    i = lax.bitcast_convert_type(d, jnp.int32)
    i = jnp.full((L,), 0x5F3759DF, jnp.int32) - (i >> 1)
    y = lax.bitcast_convert_type(i, _f32)
    for _ in range(3):
        y = y * (1.5 - 0.5 * d * y * y)
    return y


def _sc_degree(col3, npad):
    nch = col3.shape[1]
    rpt = npad // NS
    mesh = plsc.VectorSubcoreMesh(core_axis_name="c", subcore_axis_name="s")

    @functools.partial(
        pl.kernel,
        out_type=_sds((NC, npad, L)),
        mesh=mesh,
        scratch_types=[
            pltpu.VMEM((nch, CHUNK), jnp.int32),
            pltpu.VMEM((CHUNK, L), _f32),
            pltpu.VMEM((rpt, L), _f32),
            pltpu.VMEM_SHARED((npad, L), _f32),
            pltpu.SemaphoreType.DMA,
        ],
        compiler_params=pltpu.CompilerParams(use_tc_tiling_on_sc=False),
    )
    def deg(col_hbm, out_hbm, col_v, one_v, z_v, acc_sh, sem):
        cid = lax.axis_index("c")
        sid = lax.axis_index("s")
        wid = sid * NC + cid

        pltpu.sync_copy(col_hbm.at[wid], col_v)

        zero = jnp.zeros((L,), _f32)
        one = jnp.ones((L,), _f32)

        @pl.loop(0, CHUNK)
        def _(i):
            one_v[i] = one

        @pl.loop(0, rpt)
        def _(i):
            z_v[i] = zero

        pltpu.sync_copy(z_v, acc_sh.at[pl.ds(sid * rpt, rpt)])
        plsc.subcore_barrier()

        @pl.loop(0, nch)
        def _(j):
            pltpu.async_copy(one_v, acc_sh.at[col_v.at[j]], sem, add=True)

        @pl.loop(0, nch)
        def _(j):
            pltpu.make_async_copy(one_v, acc_sh.at[col_v.at[j]], sem).wait()

        plsc.subcore_barrier()
        pltpu.sync_copy(acc_sh.at[pl.ds(sid * rpt, rpt)],
                        out_hbm.at[cid, pl.ds(sid * rpt, rpt)])

    return deg(col3)


def _sc_hop(xw, degp, s_prev, row3, col3, npad):
    nch = row3.shape[1]
    rpt = npad // NS
    hop2 = s_prev is not None
    mesh = plsc.VectorSubcoreMesh(core_axis_name="c", subcore_axis_name="s")

    scratch = [
        pltpu.VMEM((nch, CHUNK), jnp.int32),
        pltpu.VMEM((nch, CHUNK), jnp.int32),
        [pltpu.VMEM((CHUNK, L), _f32)] * 4,
        pltpu.VMEM((rpt, L), _f32),
        pltpu.VMEM((rpt, L), _f32),
        pltpu.VMEM((rpt, L), _f32),
        pltpu.VMEM((rpt, L), _f32),
        pltpu.VMEM_SHARED((npad, L), _f32),
        pltpu.VMEM_SHARED((npad, L), _f32),
        [pltpu.SemaphoreType.DMA] * 4,
        [pltpu.SemaphoreType.DMA] * 4,
    ]
    if hop2:
        scratch.insert(7, pltpu.VMEM((rpt, L), _f32))
        scratch.insert(8, pltpu.VMEM((rpt, L), _f32))

    def body(refs):
        if hop2:
            (xw_hbm, dp_hbm, sp_hbm, row_hbm, col_hbm, out_hbm, row_v,
             col_v, msg, a_v, p0_v, p1_v, g_v, q0_v, q1_v, g_sh, acc_sh,
             gsem, ssem) = refs
        else:
            (xw_hbm, dp_hbm, row_hbm, col_hbm, out_hbm, row_v, col_v,
             msg, a_v, p0_v, p1_v, g_v, g_sh, acc_sh, gsem, ssem) = refs

        cid = lax.axis_index("c")
        sid = lax.axis_index("s")
        wid = sid * NC + cid
        base = sid * rpt

        pltpu.sync_copy(row_hbm.at[wid], row_v)
        pltpu.sync_copy(col_hbm.at[wid], col_v)
        pltpu.sync_copy(xw_hbm.at[pl.ds(base, rpt)], a_v)
        pltpu.sync_copy(dp_hbm.at[0, pl.ds(base, rpt)], p0_v)
        pltpu.sync_copy(dp_hbm.at[1, pl.ds(base, rpt)], p1_v)
        if hop2:
            pltpu.sync_copy(sp_hbm.at[0, pl.ds(base, rpt)], q0_v)
            pltpu.sync_copy(sp_hbm.at[1, pl.ds(base, rpt)], q1_v)

        @pl.loop(0, rpt)
        def _(i):
            dv = _rsqrt_sc(1.0 + p0_v[i] + p1_v[i])
            g = dv * a_v[i]
            if hop2:
                g = dv * dv * (q0_v[i] + q1_v[i] + g)
            g_v[i] = g

        pltpu.sync_copy(g_v, g_sh.at[pl.ds(base, rpt)])

        zero = jnp.zeros((L,), _f32)

        @pl.loop(0, rpt)
        def _(i):
            g_v[i] = zero

        pltpu.sync_copy(g_v, acc_sh.at[pl.ds(base, rpt)])
        plsc.subcore_barrier()

        def g_issue(b, j):
            pltpu.async_copy(g_sh.at[row_v.at[j]], msg[b], gsem[b])

        def g_wait(b, j):
            pltpu.make_async_copy(g_sh.at[row_v.at[j]], msg[b],
                                  gsem[b]).wait()

        def s_issue(b, j):
            pltpu.async_copy(msg[b], acc_sh.at[col_v.at[j]], ssem[b],
                             add=True)

        def s_wait(b, j):
            pltpu.make_async_copy(msg[b], acc_sh.at[col_v.at[j]],
                                  ssem[b]).wait()

        last = nch - 1
        g_issue(0, 0)
        g_issue(1, 1)
        g_wait(0, 0); s_issue(0, 0); g_issue(2, 2)
        g_wait(1, 1); s_issue(1, 1); g_issue(3, 3)
        g_wait(2, 2); s_issue(2, 2); s_wait(0, 0); g_issue(0, 4)
        g_wait(3, 3); s_issue(3, 3); s_wait(1, 1); g_issue(1, 5)

        @pl.loop(1, nch // 4)
        def _(jo):
            j0 = 4 * jo
            for b in range(4):
                j = j0 + b
                g_wait(b, j)
                s_issue(b, j)
                s_wait((b + 2) % 4, j - 2)
                g_issue((b + 2) % 4, jnp.minimum(j + 2, last))

        s_wait(2, nch - 2)
        s_wait(3, nch - 1)
        g_wait(0, last)
        g_wait(1, last)
        plsc.subcore_barrier()
        pltpu.sync_copy(acc_sh.at[pl.ds(base, rpt)],
                        out_hbm.at[cid, pl.ds(base, rpt)])

    kern = functools.partial(
        pl.kernel,
        out_type=_sds((NC, npad, L)),
        mesh=mesh,
        scratch_types=scratch,
        compiler_params=pltpu.CompilerParams(use_tc_tiling_on_sc=False),
    )

    if hop2:
        @kern
        def hop(*refs):
            body(refs)
        return hop(xw, degp, s_prev, row3, col3)

    @kern
    def hop(*refs):
        body(refs)
    return hop(xw, degp, row3, col3)


def _tc_project(x, w, npad):
    n = x.shape[0]

    def f(x_ref, w_ref, o_ref):
        o_ref[:n] = lax.dot_general(
            x_ref[...], w_ref[...], (((1,), (1,)), ((), ())),
            preferred_element_type=_f32)
        o_ref[n:] = jnp.zeros((npad - n, L), _f32)

    return pl.pallas_call(f, out_shape=_sds((npad, L)))(x, w)


def _tc_final(degp, xw, s1, s2, b2):
    def f(dp_ref, xw_ref, s1_ref, s2_ref, b_ref, o_ref):
        dv = lax.rsqrt(1.0 + dp_ref[0] + dp_ref[1])
        g0 = dv * xw_ref[...]
        g1 = dv * dv * (s1_ref[0] + s1_ref[1] + g0)
        h2 = dv * (s2_ref[0] + s2_ref[1] + g1)
        logits = h2 + b_ref[...]
        m = jnp.max(logits, axis=1, keepdims=True)
        e = jnp.exp(logits - m)
        z = jnp.sum(e, axis=1, keepdims=True)
        o_ref[...] = logits - m - jnp.log(z)

    return pl.pallas_call(
        f, out_shape=_sds(xw.shape))(degp, xw, s1, s2, b2)


def kernel(x, edge_index, W, b):
    n, d = x.shape
    c = W.shape[0]
    e = edge_index.shape[1]
    assert c == L

    npad = ((n + CHUNK - 1) // CHUNK) * CHUNK
    nch = (e + NW * CHUNK - 1) // (NW * CHUNK)
    nch = max(8, ((nch + 3) // 4) * 4)
    epad = NW * nch * CHUNK

    row3 = jnp.pad(edge_index[0], (0, epad - e)).reshape(NW, nch, CHUNK)
    col3 = jnp.pad(edge_index[1], (0, epad - e),
                   constant_values=n).reshape(NW, nch, CHUNK)

    degp = _sc_degree(col3, npad)
    xw = _tc_project(x, W, npad)
    s1 = _sc_hop(xw, degp, None, row3, col3, npad)
    s2 = _sc_hop(xw, degp, s1, row3, col3, npad)
    out = _tc_final(degp, xw, s1, s2, b.reshape(1, L))
    return out[:n]
